# baseline (speedup 1.0000x reference)
import functools

import jax
import jax.numpy as jnp
from jax import lax
from jax.experimental import pallas as pl
from jax.experimental.pallas import tpu as pltpu
from jax.experimental.pallas import tpu_sc as plsc

_DIM = 4096
_LEVELS = 256
_B = 4
_T = 8
_HW = 1024
_BT = _B * _T
_LANES = 16
_MAGIC = 12582912.0


def _sc_histogram(flat):
    mesh = plsc.VectorSubcoreMesh(core_axis_name="c", subcore_axis_name="s")

    @functools.partial(
        pl.kernel,
        mesh=mesh,
        out_type=jax.ShapeDtypeStruct((_BT, _LEVELS), jnp.float32),
        scratch_types=[
            pltpu.VMEM((_HW,), jnp.float32),
            pltpu.VMEM((_LEVELS,), jnp.float32),
        ],
        compiler_params=pltpu.CompilerParams(needs_layout_passes=False),
    )
    def hist_kernel(flat_hbm, out_hbm, row_v, hist_v):
        wid = lax.axis_index("s") * 2 + lax.axis_index("c")
        pltpu.sync_copy(flat_hbm.at[wid], row_v)

        zeros16 = jnp.zeros((_LANES,), jnp.float32)

        def zero_body(i, carry):
            hist_v[pl.ds(i * _LANES, _LANES)] = zeros16
            return carry

        lax.fori_loop(0, _LEVELS // _LANES, zero_body, 0)

        def max_body(i, m):
            return jnp.maximum(m, row_v[pl.ds(i * _LANES, _LANES)])

        m16 = lax.fori_loop(0, _HW // _LANES, max_body, zeros16)
        mx = m16[0]
        for lane in range(1, _LANES):
            mx = jnp.maximum(mx, m16[lane])
        mx16 = jnp.broadcast_to(mx, (_LANES,))

        ones16 = jnp.ones((_LANES,), jnp.float32)

        def hist_body(i, carry):
            x = row_v[pl.ds(i * _LANES, _LANES)]
            r = ((x / mx16) * (_LEVELS - 1.0) + _MAGIC) - _MAGIC
            r = jnp.clip(r, 0.0, _LEVELS - 1.0)
            plsc.addupdate_scatter(hist_v, [r.astype(jnp.int32)], ones16)
            return carry

        lax.fori_loop(0, _HW // _LANES, hist_body, 0)
        pltpu.sync_copy(hist_v, out_hbm.at[wid])

    return hist_kernel(flat)


def _tc_kernel(counts_ref, sw_ref, tw_ref, out_ref):
    counts = counts_ref[...]
    c_hi = jnp.floor(counts * (1.0 / 16.0))
    c_lo = counts - c_hi * 16.0
    sw = sw_ref[...].astype(jnp.bfloat16)
    dot = lambda a: jax.lax.dot_general(
        a.astype(jnp.bfloat16), sw,
        (((1,), (0,)), ((), ())),
        preferred_element_type=jnp.float32)
    w = dot(c_hi) * 16.0 + dot(c_lo)

    tw = tw_ref[...]
    m = w * jnp.concatenate([tw] * _B, axis=0)
    rows = [jnp.sum(m[b * _T:(b + 1) * _T, :], axis=0, keepdims=True)
            for b in range(_B)]
    out_ref[...] = jnp.sign(jnp.concatenate(rows, axis=0))


def kernel(data, spatial_weight, temporal_weight):
    b, t, c, h, w = data.shape
    flat = data.reshape(b * t, c * h * w)
    counts = _sc_histogram(flat)
    return pl.pallas_call(
        _tc_kernel,
        grid=(1,),
        out_shape=jax.ShapeDtypeStruct((b, _DIM), jnp.float32),
        in_specs=[
            pl.BlockSpec((_BT, _LEVELS), lambda i: (0, 0)),
            pl.BlockSpec((_LEVELS, _DIM), lambda i: (0, 0)),
            pl.BlockSpec((t, _DIM), lambda i: (0, 0)),
        ],
        out_specs=pl.BlockSpec((b, _DIM), lambda i: (0, 0)),
    )(counts, spatial_weight, temporal_weight)


# speedup vs baseline: 3.3611x; 3.3611x over previous
import jax
import jax.numpy as jnp
from jax.experimental import pallas as pl
from jax.experimental.pallas import tpu as pltpu

_DIM = 4096
_LEVELS = 256
_B = 4
_T = 8
_HW = 1024
_BT = _B * _T
_NBLK = 4
_DBLK = _DIM // _NBLK


def _encode_kernel(flat_ref, sw_ref, tw_ref, out_ref, chi_ref, clo_ref):
    @pl.when(pl.program_id(0) == 0)
    def _():
        flat = flat_ref[...]
        mx = jnp.max(flat, axis=1, keepdims=True)
        scaled = flat / mx * (_LEVELS - 1.0)
        q = jnp.clip(jnp.round(scaled), 0.0, _LEVELS - 1.0).astype(jnp.int32)

        nib = jax.lax.broadcasted_iota(jnp.int32, (1, 16, 1), 1)
        q3 = q[:, None, :]
        hi = ((q3 >> 4) == nib).astype(jnp.bfloat16)
        lo = ((q3 & 15) == nib).astype(jnp.bfloat16)
        counts3 = jax.lax.dot_general(
            hi, lo,
            (((2,), (2,)), ((0,), (0,))),
            preferred_element_type=jnp.float32)
        counts = counts3.reshape(_BT, _LEVELS)

        c_hi = jnp.floor(counts * (1.0 / 16.0))
        chi_ref[...] = c_hi.astype(jnp.bfloat16)
        clo_ref[...] = (counts - c_hi * 16.0).astype(jnp.bfloat16)

    sw = sw_ref[...].astype(jnp.bfloat16)
    dot = lambda a: jax.lax.dot_general(
        a, sw,
        (((1,), (0,)), ((), ())),
        preferred_element_type=jnp.float32)
    w = dot(chi_ref[...]) * 16.0 + dot(clo_ref[...])

    tw = tw_ref[...]
    m = w * jnp.concatenate([tw] * _B, axis=0)
    rows = [jnp.sum(m[b * _T:(b + 1) * _T, :], axis=0, keepdims=True)
            for b in range(_B)]
    out_ref[...] = jnp.sign(jnp.concatenate(rows, axis=0))


def kernel(data, spatial_weight, temporal_weight):
    b, t, c, h, w = data.shape
    flat = data.reshape(b * t, c * h * w)
    return pl.pallas_call(
        _encode_kernel,
        grid=(_NBLK,),
        out_shape=jax.ShapeDtypeStruct((b, _DIM), jnp.float32),
        in_specs=[
            pl.BlockSpec((_BT, _HW), lambda k: (0, 0)),
            pl.BlockSpec((_LEVELS, _DBLK), lambda k: (0, k)),
            pl.BlockSpec((t, _DBLK), lambda k: (0, k)),
        ],
        out_specs=pl.BlockSpec((b, _DBLK), lambda k: (0, k)),
        scratch_shapes=[
            pltpu.VMEM((_BT, _LEVELS), jnp.bfloat16),
            pltpu.VMEM((_BT, _LEVELS), jnp.bfloat16),
        ],
        compiler_params=pltpu.CompilerParams(
            dimension_semantics=("arbitrary",)),
    )(flat, spatial_weight, temporal_weight)


# speedup vs baseline: 3.8117x; 1.1341x over previous
import jax
import jax.numpy as jnp
from jax.experimental import pallas as pl
from jax.experimental.pallas import tpu as pltpu

_DIM = 4096
_LEVELS = 256
_B = 4
_T = 8
_HW = 1024
_BT = _B * _T
_NBLK = 2
_DBLK = _DIM // _NBLK


def _encode_kernel(flat_ref, sw_ref, tw_ref, out_ref, chi_ref, clo_ref):
    @pl.when(pl.program_id(0) == 0)
    def _():
        flat = flat_ref[...]
        mx = jnp.max(flat, axis=1, keepdims=True)
        scaled = flat / mx * (_LEVELS - 1.0)
        q = jnp.clip(jnp.round(scaled), 0.0, _LEVELS - 1.0).astype(jnp.int32)

        nib = jax.lax.broadcasted_iota(jnp.int32, (1, 16, 1), 1)
        q3 = q[:, None, :]
        hi = ((q3 >> 4) == nib).astype(jnp.bfloat16)
        lo = ((q3 & 15) == nib).astype(jnp.bfloat16)
        counts3 = jax.lax.dot_general(
            hi, lo,
            (((2,), (2,)), ((0,), (0,))),
            preferred_element_type=jnp.float32)
        counts = counts3.reshape(_BT, _LEVELS)

        c_hi = jnp.floor(counts * (1.0 / 16.0))
        chi_ref[...] = c_hi.astype(jnp.bfloat16)
        clo_ref[...] = (counts - c_hi * 16.0).astype(jnp.bfloat16)

    sw = sw_ref[...].astype(jnp.bfloat16)
    dot = lambda a: jax.lax.dot_general(
        a, sw,
        (((1,), (0,)), ((), ())),
        preferred_element_type=jnp.float32)
    w = dot(chi_ref[...]) * 16.0 + dot(clo_ref[...])

    tw = tw_ref[...]
    m = w * jnp.concatenate([tw] * _B, axis=0)
    rows = [jnp.sum(m[b * _T:(b + 1) * _T, :], axis=0, keepdims=True)
            for b in range(_B)]
    out_ref[...] = jnp.sign(jnp.concatenate(rows, axis=0))


def kernel(data, spatial_weight, temporal_weight):
    b, t, c, h, w = data.shape
    flat = data.reshape(b * t, c * h * w)
    return pl.pallas_call(
        _encode_kernel,
        grid=(_NBLK,),
        out_shape=jax.ShapeDtypeStruct((b, _DIM), jnp.float32),
        in_specs=[
            pl.BlockSpec((_BT, _HW), lambda k: (0, 0)),
            pl.BlockSpec((_LEVELS, _DBLK), lambda k: (0, k)),
            pl.BlockSpec((t, _DBLK), lambda k: (0, k)),
        ],
        out_specs=pl.BlockSpec((b, _DBLK), lambda k: (0, k)),
        scratch_shapes=[
            pltpu.VMEM((_BT, _LEVELS), jnp.bfloat16),
            pltpu.VMEM((_BT, _LEVELS), jnp.bfloat16),
        ],
        compiler_params=pltpu.CompilerParams(
            dimension_semantics=("arbitrary",)),
    )(flat, spatial_weight, temporal_weight)


# speedup vs baseline: 3.9838x; 1.0451x over previous
import jax
import jax.numpy as jnp
from jax.experimental import pallas as pl
from jax.experimental.pallas import tpu as pltpu

_DIM = 4096
_LEVELS = 256
_B = 4
_T = 8
_HW = 1024
_BT = _B * _T
_HALF = _LEVELS // 2


def _encode_kernel(flat_ref, sw_hbm, tw_ref, out_ref, sw_vmem, sem0, sem1):
    copy0 = pltpu.make_async_copy(
        sw_hbm.at[pl.ds(0, _HALF), :], sw_vmem.at[pl.ds(0, _HALF), :], sem0)
    copy1 = pltpu.make_async_copy(
        sw_hbm.at[pl.ds(_HALF, _HALF), :], sw_vmem.at[pl.ds(_HALF, _HALF), :],
        sem1)
    copy0.start()
    copy1.start()

    flat = flat_ref[...]
    mx = jnp.max(flat, axis=1, keepdims=True)
    scaled = flat / mx * (_LEVELS - 1.0)
    q = jnp.clip(jnp.round(scaled), 0.0, _LEVELS - 1.0).astype(jnp.int32)

    nib = jax.lax.broadcasted_iota(jnp.int32, (1, 16, 1), 1)
    q3 = q[:, None, :]
    hi = ((q3 >> 4) == nib).astype(jnp.bfloat16)
    lo = ((q3 & 15) == nib).astype(jnp.bfloat16)
    counts3 = jax.lax.dot_general(
        hi, lo,
        (((2,), (2,)), ((0,), (0,))),
        preferred_element_type=jnp.float32)
    counts = counts3.reshape(_BT, _LEVELS)

    c_hi_f = jnp.floor(counts * (1.0 / 16.0))
    c_hi = c_hi_f.astype(jnp.bfloat16)
    c_lo = (counts - c_hi_f * 16.0).astype(jnp.bfloat16)

    def dot(a, b):
        return jax.lax.dot_general(
            a, b, (((1,), (0,)), ((), ())),
            preferred_element_type=jnp.float32)

    copy0.wait()
    sw0 = sw_vmem[0:_HALF, :].astype(jnp.bfloat16)
    w = dot(c_hi[:, 0:_HALF], sw0) * 16.0 + dot(c_lo[:, 0:_HALF], sw0)
    copy1.wait()
    sw1 = sw_vmem[_HALF:_LEVELS, :].astype(jnp.bfloat16)
    w = w + dot(c_hi[:, _HALF:], sw1) * 16.0 + dot(c_lo[:, _HALF:], sw1)

    tw = tw_ref[...]
    m = w * jnp.concatenate([tw] * _B, axis=0)
    rows = [jnp.sum(m[b * _T:(b + 1) * _T, :], axis=0, keepdims=True)
            for b in range(_B)]
    out_ref[...] = jnp.sign(jnp.concatenate(rows, axis=0))


def kernel(data, spatial_weight, temporal_weight):
    b, t, c, h, w = data.shape
    flat = data.reshape(b * t, c * h * w)
    return pl.pallas_call(
        _encode_kernel,
        grid=(1,),
        out_shape=jax.ShapeDtypeStruct((b, _DIM), jnp.float32),
        in_specs=[
            pl.BlockSpec((_BT, _HW), lambda i: (0, 0)),
            pl.BlockSpec(memory_space=pl.ANY),
            pl.BlockSpec((t, _DIM), lambda i: (0, 0)),
        ],
        out_specs=pl.BlockSpec((b, _DIM), lambda i: (0, 0)),
        scratch_shapes=[
            pltpu.VMEM((_LEVELS, _DIM), jnp.float32),
            pltpu.SemaphoreType.DMA,
            pltpu.SemaphoreType.DMA,
        ],
    )(flat, spatial_weight, temporal_weight)
